# Initial kernel scaffold; baseline (speedup 1.0000x reference)
import jax
import jax.numpy as jnp
from jax.experimental import pallas as pl


def kernel(x, edge_index, edge_attr, rev_edge_index, W_i, b_i, W_h, b_h, W_o, b_o):
    raise NotImplementedError("write your pallas kernel here")


# speedup vs baseline: 1.3365x; 1.3365x over previous
import functools

import jax
import jax.numpy as jnp
from jax import lax
from jax.experimental import pallas as pl
from jax.experimental.pallas import tpu as pltpu
from jax.experimental.pallas import tpu_sc as plsc

F32 = jnp.float32
I32 = jnp.int32

CH = 80
CROWS = 80


def _tc_linear(a, w, b, block_rows):
    R, K = a.shape

    def body(a_ref, w_ref, b_ref, o_ref):
        o = lax.dot_general(a_ref[...], w_ref[...], (((1,), (1,)), ((), ())),
                            preferred_element_type=F32)
        o_ref[...] = o + b_ref[...]

    return pl.pallas_call(
        body,
        grid=(R // block_rows,),
        in_specs=[
            pl.BlockSpec((block_rows, K), lambda i: (i, 0)),
            pl.BlockSpec((128, K), lambda i: (0, 0)),
            pl.BlockSpec((1, 128), lambda i: (0, 0)),
        ],
        out_specs=pl.BlockSpec((block_rows, 128), lambda i: (i, 0)),
        out_shape=jax.ShapeDtypeStruct((R, 128), F32),
    )(a, w, b.reshape(1, 128))


def _tc_relu_matmul(p, w_h, block_rows):
    E = p.shape[0]

    def body(p_ref, w_ref, o_ref):
        h = jnp.maximum(p_ref[...], 0.0)
        o_ref[...] = lax.dot_general(h, w_ref[...], (((1,), (1,)), ((), ())),
                                     preferred_element_type=F32)

    return pl.pallas_call(
        body,
        grid=(E // block_rows,),
        in_specs=[
            pl.BlockSpec((block_rows, 128), lambda i: (i, 0)),
            pl.BlockSpec((128, 128), lambda i: (0, 0)),
        ],
        out_specs=pl.BlockSpec((block_rows, 128), lambda i: (i, 0)),
        out_shape=jax.ShapeDtypeStruct((E, 128), F32),
    )(p, w_h)


def _tc_final(x, mn2, w_o, b_o, block_rows):
    N = x.shape[0]

    def body(x_ref, m_ref, w_ref, b_ref, o_ref):
        xb = x_ref[...]
        m = m_ref[0] + m_ref[1]
        iso = jnp.sum(m, axis=1, keepdims=True) == 0.0
        m = jnp.where(iso, xb, m)
        w = w_ref[...]
        o = lax.dot_general(xb, w[:, :128], (((1,), (1,)), ((), ())),
                            preferred_element_type=F32)
        o = o + lax.dot_general(m, w[:, 128:], (((1,), (1,)), ((), ())),
                                preferred_element_type=F32)
        o_ref[...] = jnp.maximum(o + b_ref[...], 0.0)

    return pl.pallas_call(
        body,
        grid=(N // block_rows,),
        in_specs=[
            pl.BlockSpec((block_rows, 128), lambda i: (i, 0)),
            pl.BlockSpec((2, block_rows, 128), lambda i: (0, i, 0)),
            pl.BlockSpec((128, 256), lambda i: (0, 0)),
            pl.BlockSpec((1, 128), lambda i: (0, 0)),
        ],
        out_specs=pl.BlockSpec((block_rows, 128), lambda i: (i, 0)),
        out_shape=jax.ShapeDtypeStruct((N, 128), F32),
    )(x, mn2, w_o, b_o.reshape(1, 128))


def _sc_mesh():
    return plsc.VectorSubcoreMesh(core_axis_name="c", subcore_axis_name="s")


def _for_tile_rows(s, n_rows, fn):
    base = (n_rows // 16) // 8 * 8
    fn(s * base, base)
    tail = n_rows - 16 * base
    if tail:
        @pl.when(s == 15)
        def _():
            fn(16 * base, tail)


def _relu_rows(buf, n):
    def row(r, carry):
        for q in range(8):
            sl = pl.ds(q * 16, 16)
            buf[r, sl] = jnp.maximum(buf[r, sl], 0.0)
        return carry

    lax.fori_loop(0, n, row, 0)


def _add_rows(dst, src, n):
    def row(r, carry):
        for q in range(8):
            sl = pl.ds(q * 16, 16)
            dst[r, sl] = dst[r, sl] + src[r, sl]
        return carry

    lax.fori_loop(0, n, row, 0)


def _sc_h0(xp, eh, src, N, E):
    ep = E // 32
    nch = ep // CH

    @functools.partial(
        pl.kernel,
        out_type=jax.ShapeDtypeStruct((E, 128), F32),
        mesh=_sc_mesh(),
        scratch_types=[
            pltpu.VMEM_SHARED((N, 128), F32),
            pltpu.VMEM((CH,), I32),
            pltpu.VMEM((CH, 128), F32),
            pltpu.VMEM((CH, 128), F32),
        ],
    )
    def k(xp_hbm, eh_hbm, src_hbm, h0_out, stage, iv, gbuf, pbuf):
        c = lax.axis_index("c")
        s = lax.axis_index("s")
        wid = c * 16 + s
        _for_tile_rows(s, N, lambda r0, nr: pltpu.sync_copy(
            xp_hbm.at[pl.ds(r0, nr)], stage.at[pl.ds(r0, nr)]))
        plsc.subcore_barrier()

        def chunk(j, carry):
            off = wid * ep + j * CH
            pltpu.sync_copy(src_hbm.at[pl.ds(off, CH)], iv)
            pltpu.sync_copy(eh_hbm.at[pl.ds(off, CH)], pbuf)
            pltpu.sync_copy(stage.at[iv], gbuf)
            _add_rows(pbuf, gbuf, CH)
            pltpu.sync_copy(pbuf, h0_out.at[pl.ds(off, CH)])
            return carry

        lax.fori_loop(0, nch, chunk, 0)

    return k(xp, eh, src)


def _sc_scatter(q, src, dst, zeros_n, N, E, relu):
    ep = E // 32
    nch = ep // CH
    have_src = src is not None

    @functools.partial(
        pl.kernel,
        out_type=jax.ShapeDtypeStruct((2, N, 128), F32),
        mesh=_sc_mesh(),
        scratch_types=[
            pltpu.VMEM_SHARED((N, 128), F32),
            pltpu.VMEM((CH,), I32),
            pltpu.VMEM((CH,), I32),
            pltpu.VMEM((CH, 128), F32),
        ],
    )
    def k(q_hbm, src_hbm, dst_hbm, z_hbm, part_out, acc, iv1, iv2, gbuf):
        c = lax.axis_index("c")
        s = lax.axis_index("s")
        wid = c * 16 + s
        _for_tile_rows(s, N, lambda r0, nr: pltpu.sync_copy(
            z_hbm.at[pl.ds(r0, nr)], acc.at[pl.ds(r0, nr)]))
        plsc.subcore_barrier()

        def chunk(j, carry):
            off = wid * ep + j * CH
            pltpu.sync_copy(dst_hbm.at[pl.ds(off, CH)], iv2)
            if have_src:
                pltpu.sync_copy(src_hbm.at[pl.ds(off, CH)], iv1)
                pltpu.sync_copy(q_hbm.at[iv1], gbuf)
            else:
                pltpu.sync_copy(q_hbm.at[pl.ds(off, CH)], gbuf)
            if relu:
                _relu_rows(gbuf, CH)
            pltpu.sync_copy(gbuf, acc.at[iv2], add=True)
            return carry

        lax.fori_loop(0, nch, chunk, 0)
        plsc.subcore_barrier()
        _for_tile_rows(s, N, lambda r0, nr: pltpu.sync_copy(
            acc.at[pl.ds(r0, nr)], part_out.at[c, pl.ds(r0, nr)]))

    if have_src:
        return k(q, src, dst, zeros_n)
    return k(q, dst, dst, zeros_n)


def _sc_assemble(part, q, h0, b_h, src, rev, N, E):
    ep = E // 32
    nch = ep // CH

    @functools.partial(
        pl.kernel,
        out_type=jax.ShapeDtypeStruct((E, 128), F32),
        mesh=_sc_mesh(),
        scratch_types=[
            pltpu.VMEM_SHARED((N, 128), F32),
            pltpu.VMEM((CH,), I32),
            pltpu.VMEM((CH,), I32),
            pltpu.VMEM((CH, 128), F32),
            pltpu.VMEM((CH, 128), F32),
            pltpu.VMEM((CH, 128), F32),
            pltpu.VMEM((128,), F32),
        ],
    )
    def k(part_hbm, q_hbm, h0_hbm, bh_hbm, src_hbm, rev_hbm, p_out,
          stage, iv1, iv2, gbuf, pbuf, rbuf, bias_v):
        c = lax.axis_index("c")
        s = lax.axis_index("s")
        wid = c * 16 + s
        pltpu.sync_copy(bh_hbm, bias_v)

        def crows(r0, nr):
            def combine(rr, rows):
                pltpu.sync_copy(part_hbm.at[0, pl.ds(rr, rows)],
                                pbuf.at[pl.ds(0, rows)])
                pltpu.sync_copy(part_hbm.at[1, pl.ds(rr, rows)],
                                gbuf.at[pl.ds(0, rows)])
                _add_rows(pbuf, gbuf, rows)
                pltpu.sync_copy(pbuf.at[pl.ds(0, rows)],
                                stage.at[pl.ds(rr, rows)])

            def cchunk(m, carry):
                combine(r0 + m * CROWS, CROWS)
                return carry

            lax.fori_loop(0, nr // CROWS, cchunk, 0)
            rem = nr % CROWS
            if rem:
                combine(r0 + nr - rem, rem)

        _for_tile_rows(s, N, crows)
        plsc.subcore_barrier()

        bias = tuple(bias_v[pl.ds(q * 16, 16)] for q in range(8))

        def chunk(j, carry):
            off = wid * ep + j * CH
            pltpu.sync_copy(src_hbm.at[pl.ds(off, CH)], iv1)
            pltpu.sync_copy(rev_hbm.at[pl.ds(off, CH)], iv2)
            pltpu.sync_copy(h0_hbm.at[pl.ds(off, CH)], pbuf)
            pltpu.sync_copy(stage.at[iv1], gbuf)
            pltpu.sync_copy(q_hbm.at[iv2], rbuf)

            def row(r, carry2):
                for q in range(8):
                    sl = pl.ds(q * 16, 16)
                    pbuf[r, sl] = (pbuf[r, sl] + bias[q] + gbuf[r, sl]
                                   - rbuf[r, sl])
                return carry2

            lax.fori_loop(0, CH, row, 0)
            pltpu.sync_copy(pbuf, p_out.at[pl.ds(off, CH)])
            return carry

        lax.fori_loop(0, nch, chunk, 0)

    return k(part, q, h0, b_h, src, rev)


def kernel(x, edge_index, edge_attr, rev_edge_index, W_i, b_i, W_h, b_h,
           W_o, b_o):
    N, DF = x.shape
    E = edge_attr.shape[0]
    depth = 5

    src = edge_index[0]
    dst = edge_index[1]
    zeros_n = jnp.zeros((N, 128), F32)

    W_ix = W_i[:, :DF]
    W_ie = W_i[:, DF:]

    xp = _tc_linear(x, W_ix, b_i, 1000)
    eh = _tc_linear(edge_attr, W_ie, jnp.zeros((128,), F32), 2000)
    h0 = _sc_h0(xp, eh, src, N, E)

    p = h0
    for _ in range(1, depth):
        q = _tc_relu_matmul(p, W_h, 2000)
        part = _sc_scatter(q, src, dst, zeros_n, N, E, relu=False)
        p = _sc_assemble(part, q, h0, b_h, src, rev_edge_index, N, E)

    mn2 = _sc_scatter(p, None, dst, zeros_n, N, E, relu=True)
    return _tc_final(x, mn2, W_o, b_o, 1000)


# speedup vs baseline: 1.5126x; 1.1317x over previous
import functools

import jax
import jax.numpy as jnp
from jax import lax
from jax.experimental import pallas as pl
from jax.experimental.pallas import tpu as pltpu
from jax.experimental.pallas import tpu_sc as plsc

F32 = jnp.float32
I32 = jnp.int32

SUPER = 2000
NBUF = 3


def _tc_linear(a, w, b, block_rows):
    R, K = a.shape

    def body(a_ref, w_ref, b_ref, o_ref):
        o = lax.dot_general(a_ref[...], w_ref[...], (((1,), (1,)), ((), ())),
                            preferred_element_type=F32)
        o_ref[...] = o + b_ref[...]

    return pl.pallas_call(
        body,
        grid=(R // block_rows,),
        in_specs=[
            pl.BlockSpec((block_rows, K), lambda i: (i, 0)),
            pl.BlockSpec((128, K), lambda i: (0, 0)),
            pl.BlockSpec((1, 128), lambda i: (0, 0)),
        ],
        out_specs=pl.BlockSpec((block_rows, 128), lambda i: (i, 0)),
        out_shape=jax.ShapeDtypeStruct((R, 128), F32),
    )(a, w, b.reshape(1, 128))


def _tc_relu_matmul(h0, d, b_h, w_h, block_rows):
    E = h0.shape[0]

    if d is None:
        def body(h_ref, w_ref, o_ref):
            h = jnp.maximum(h_ref[...], 0.0)
            o_ref[...] = lax.dot_general(
                h, w_ref[...], (((1,), (1,)), ((), ())),
                preferred_element_type=F32)

        in_specs = [
            pl.BlockSpec((block_rows, 128), lambda i: (i, 0)),
            pl.BlockSpec((128, 128), lambda i: (0, 0)),
        ]
        args = (h0, w_h)
    else:
        def body(h_ref, d_ref, b_ref, w_ref, o_ref):
            h = jnp.maximum(h_ref[...] + d_ref[...] + b_ref[...], 0.0)
            o_ref[...] = lax.dot_general(
                h, w_ref[...], (((1,), (1,)), ((), ())),
                preferred_element_type=F32)

        in_specs = [
            pl.BlockSpec((block_rows, 128), lambda i: (i, 0)),
            pl.BlockSpec((block_rows, 128), lambda i: (i, 0)),
            pl.BlockSpec((1, 128), lambda i: (0, 0)),
            pl.BlockSpec((128, 128), lambda i: (0, 0)),
        ]
        args = (h0, d, b_h.reshape(1, 128), w_h)

    return pl.pallas_call(
        body,
        grid=(E // block_rows,),
        in_specs=in_specs,
        out_specs=pl.BlockSpec((block_rows, 128), lambda i: (i, 0)),
        out_shape=jax.ShapeDtypeStruct((E, 128), F32),
    )(*args)


def _tc_final(x, mn2, w_o, b_o, block_rows):
    N = x.shape[0]

    def body(x_ref, m_ref, w_ref, b_ref, o_ref):
        xb = x_ref[...]
        m = m_ref[0] + m_ref[1]
        iso = jnp.sum(m, axis=1, keepdims=True) == 0.0
        m = jnp.where(iso, xb, m)
        w = w_ref[...]
        o = lax.dot_general(xb, w[:, :128], (((1,), (1,)), ((), ())),
                            preferred_element_type=F32)
        o = o + lax.dot_general(m, w[:, 128:], (((1,), (1,)), ((), ())),
                                preferred_element_type=F32)
        o_ref[...] = jnp.maximum(o + b_ref[...], 0.0)

    return pl.pallas_call(
        body,
        grid=(N // block_rows,),
        in_specs=[
            pl.BlockSpec((block_rows, 128), lambda i: (i, 0)),
            pl.BlockSpec((2, block_rows, 128), lambda i: (0, i, 0)),
            pl.BlockSpec((128, 256), lambda i: (0, 0)),
            pl.BlockSpec((1, 128), lambda i: (0, 0)),
        ],
        out_specs=pl.BlockSpec((block_rows, 128), lambda i: (i, 0)),
        out_shape=jax.ShapeDtypeStruct((N, 128), F32),
    )(x, mn2, w_o, b_o.reshape(1, 128))


def _sc_mesh():
    return plsc.VectorSubcoreMesh(core_axis_name="c", subcore_axis_name="s")


def _for_tile_rows(s, n_rows, fn):
    base = (n_rows // 16) // 8 * 8
    fn(s * base, base)
    tail = n_rows - 16 * base
    if tail:
        @pl.when(s == 15)
        def _():
            fn(16 * base, tail)


def _sc_h0(xp, eh, src, N, E):
    CH = 40
    ep = E // 32
    nsc = ep // SUPER
    npc = SUPER // CH

    @functools.partial(
        pl.kernel,
        out_type=jax.ShapeDtypeStruct((E, 128), F32),
        mesh=_sc_mesh(),
        scratch_types=[
            pltpu.VMEM_SHARED((N, 128), F32),
            pltpu.VMEM((2 * SUPER,), I32),
            pltpu.VMEM((NBUF, CH, 128), F32),
            pltpu.VMEM((NBUF, CH, 128), F32),
            pltpu.SemaphoreType.DMA((2,)),
            pltpu.SemaphoreType.DMA((NBUF,)),
            pltpu.SemaphoreType.DMA((NBUF,)),
            pltpu.SemaphoreType.DMA((NBUF,)),
        ],
    )
    def k(xp_hbm, eh_hbm, src_hbm, h0_out, stage, iv, ebuf, gbuf,
          sem_i, sem_l, sem_g, sem_w):
        c = lax.axis_index("c")
        s = lax.axis_index("s")
        wid = c * 16 + s
        base = wid * ep
        _for_tile_rows(s, N, lambda r0, nr: pltpu.sync_copy(
            xp_hbm.at[pl.ds(r0, nr)], stage.at[pl.ds(r0, nr)]))
        plsc.subcore_barrier()

        def idx_load(sc, slot):
            pltpu.async_copy(src_hbm.at[pl.ds(base + sc * SUPER, SUPER)],
                             iv.at[pl.ds(slot * SUPER, SUPER)],
                             sem_i.at[slot])

        def idx_wait(slot):
            pltpu.make_async_copy(
                src_hbm.at[pl.ds(base, SUPER)],
                iv.at[pl.ds(slot * SUPER, SUPER)], sem_i.at[slot]).wait()

        def issue(sc, slot, w, b):
            off = base + sc * SUPER + w * CH
            ivs = iv.at[pl.ds(slot * SUPER + w * CH, CH)]
            pltpu.async_copy(eh_hbm.at[pl.ds(off, CH)], ebuf.at[b],
                             sem_l.at[b])
            pltpu.async_copy(stage.at[ivs], gbuf.at[b], sem_g.at[b])

        def wait_reads(b):
            pltpu.make_async_copy(eh_hbm.at[pl.ds(0, CH)], ebuf.at[b],
                                  sem_l.at[b]).wait()
            pltpu.make_async_copy(stage.at[pl.ds(0, CH)], gbuf.at[b],
                                  sem_g.at[b]).wait()

        def drain_write(b):
            pltpu.make_async_copy(ebuf.at[b], h0_out.at[pl.ds(0, CH)],
                                  sem_w.at[b]).wait()

        idx_load(0, 0)
        for sc in range(nsc):
            slot = sc % 2
            idx_wait(slot)
            if sc + 1 < nsc:
                idx_load(sc + 1, 1 - slot)
            issue(sc, slot, 0, 0)
            issue(sc, slot, 1, 1)

            def chunk(w, carry):
                b = w % NBUF
                wait_reads(b)

                @pl.when(jnp.logical_and(w >= 1, w < npc - 2))
                def _():
                    drain_write((w + 2) % NBUF)
                    issue(sc, slot, w + 2, (w + 2) % NBUF)

                @pl.when(w == 0)
                def _():
                    issue(sc, slot, 2, 2 % NBUF)

                def row(r, carry2):
                    for q in range(8):
                        sl = pl.ds(q * 16, 16)
                        ebuf[b, r, sl] = ebuf[b, r, sl] + gbuf[b, r, sl]
                    return carry2

                lax.fori_loop(0, CH, row, 0)
                off = base + sc * SUPER + w * CH
                pltpu.async_copy(ebuf.at[b], h0_out.at[pl.ds(off, CH)],
                                 sem_w.at[b])
                return carry

            lax.fori_loop(0, npc, chunk, 0)
            for b in range(NBUF):
                drain_write(b)

    return k(xp, eh, src)


def _sc_scatter(q, src, dst, h0, b_h, zeros_n, N, E, relu_h0d):
    CH = 80
    nbuf = 2 if relu_h0d else NBUF
    ep = E // 32
    nsc = ep // SUPER
    npc = SUPER // CH

    @functools.partial(
        pl.kernel,
        out_type=jax.ShapeDtypeStruct((2, N, 128), F32),
        mesh=_sc_mesh(),
        scratch_types=[
            pltpu.VMEM_SHARED((N, 128), F32),
            pltpu.VMEM((2 * SUPER,), I32),
            pltpu.VMEM((2 * SUPER,), I32),
            pltpu.VMEM((nbuf, 1, CH), I32),
            pltpu.VMEM((nbuf, CH, 128), F32),
            pltpu.VMEM((nbuf, CH, 128), F32) if relu_h0d else
            pltpu.VMEM((1, 1, 16), F32),
            pltpu.VMEM((128,), F32),
            pltpu.SemaphoreType.DMA((2,)),
            pltpu.SemaphoreType.DMA((nbuf,)),
            pltpu.SemaphoreType.DMA((nbuf,)),
            pltpu.SemaphoreType.DMA((nbuf,)),
        ],
    )
    def k(q_hbm, src_hbm, dst_hbm, h0_hbm, bh_hbm, z_hbm, part_out,
          acc, iv1, iv2, ivs, gbuf, hbuf, bias_v,
          sem_i, sem_g, sem_h, sem_s):
        c = lax.axis_index("c")
        s = lax.axis_index("s")
        wid = c * 16 + s
        base = wid * ep
        _for_tile_rows(s, N, lambda r0, nr: pltpu.sync_copy(
            z_hbm.at[pl.ds(r0, nr)], acc.at[pl.ds(r0, nr)]))
        if relu_h0d:
            pltpu.sync_copy(bh_hbm, bias_v)
        plsc.subcore_barrier()
        if relu_h0d:
            bias = tuple(bias_v[pl.ds(q * 16, 16)] for q in range(8))

        def idx_load(sc, slot):
            o = pl.ds(base + sc * SUPER, SUPER)
            if not relu_h0d:
                pltpu.async_copy(src_hbm.at[o],
                                 iv1.at[pl.ds(slot * SUPER, SUPER)],
                                 sem_i.at[slot])
            pltpu.async_copy(dst_hbm.at[o],
                             iv2.at[pl.ds(slot * SUPER, SUPER)],
                             sem_i.at[slot])

        def idx_wait(slot):
            if not relu_h0d:
                pltpu.make_async_copy(
                    src_hbm.at[pl.ds(base, SUPER)],
                    iv1.at[pl.ds(slot * SUPER, SUPER)],
                    sem_i.at[slot]).wait()
            pltpu.make_async_copy(
                dst_hbm.at[pl.ds(base, SUPER)],
                iv2.at[pl.ds(slot * SUPER, SUPER)], sem_i.at[slot]).wait()

        def issue(sc, slot, w, b):
            off = base + sc * SUPER + w * CH
            if relu_h0d:
                pltpu.async_copy(q_hbm.at[pl.ds(off, CH)], gbuf.at[b],
                                 sem_g.at[b])
                pltpu.async_copy(h0_hbm.at[pl.ds(off, CH)], hbuf.at[b],
                                 sem_h.at[b])
            else:
                ivg = iv1.at[pl.ds(slot * SUPER + w * CH, CH)]
                pltpu.async_copy(q_hbm.at[ivg], gbuf.at[b], sem_g.at[b])
            for t in range(CH // 16):
                ivs[b, 0, pl.ds(16 * t, 16)] = iv2[
                    pl.ds(slot * SUPER + w * CH + 16 * t, 16)]

        def wait_reads(b):
            pltpu.make_async_copy(q_hbm.at[pl.ds(0, CH)], gbuf.at[b],
                                  sem_g.at[b]).wait()
            if relu_h0d:
                pltpu.make_async_copy(h0_hbm.at[pl.ds(0, CH)], hbuf.at[b],
                                      sem_h.at[b]).wait()

        def drain_scatter(b):
            pltpu.make_async_copy(gbuf.at[b], acc.at[pl.ds(0, CH)],
                                  sem_s.at[b]).wait()

        look = nbuf - 1
        idx_load(0, 0)
        for sc in range(nsc):
            slot = sc % 2
            idx_wait(slot)
            if sc + 1 < nsc:
                idx_load(sc + 1, 1 - slot)
            for t in range(look):
                issue(sc, slot, t, t)

            def chunk(w, carry):
                b = w % nbuf
                wait_reads(b)

                @pl.when(jnp.logical_and(w >= 1, w < npc - look))
                def _():
                    drain_scatter((w + look) % nbuf)
                    issue(sc, slot, w + look, (w + look) % nbuf)

                @pl.when(w == 0)
                def _():
                    issue(sc, slot, look, look % nbuf)

                if relu_h0d:
                    def row(r, carry2):
                        for q in range(8):
                            sl = pl.ds(q * 16, 16)
                            gbuf[b, r, sl] = jnp.maximum(
                                gbuf[b, r, sl] + hbuf[b, r, sl] + bias[q],
                                0.0)
                        return carry2

                    lax.fori_loop(0, CH, row, 0)
                pltpu.async_copy(gbuf.at[b], acc.at[ivs.at[b, 0]],
                                 sem_s.at[b], add=True)
                return carry

            lax.fori_loop(0, npc, chunk, 0)
            for b in range(nbuf):
                drain_scatter(b)

        plsc.subcore_barrier()
        _for_tile_rows(s, N, lambda r0, nr: pltpu.sync_copy(
            acc.at[pl.ds(r0, nr)], part_out.at[c, pl.ds(r0, nr)]))

    return k(q, src, dst, h0, b_h, zeros_n)


def _sc_assemble(part, q, src, rev, N, E):
    CH = 40
    CROWS = 40
    ep = E // 32
    nsc = ep // SUPER
    npc = SUPER // CH

    @functools.partial(
        pl.kernel,
        out_type=jax.ShapeDtypeStruct((E, 128), F32),
        mesh=_sc_mesh(),
        scratch_types=[
            pltpu.VMEM_SHARED((N, 128), F32),
            pltpu.VMEM((2 * SUPER,), I32),
            pltpu.VMEM((2 * SUPER,), I32),
            pltpu.VMEM((NBUF, CH, 128), F32),
            pltpu.VMEM((NBUF, CH, 128), F32),
            pltpu.SemaphoreType.DMA((2,)),
            pltpu.SemaphoreType.DMA((NBUF,)),
            pltpu.SemaphoreType.DMA((NBUF,)),
            pltpu.SemaphoreType.DMA((NBUF,)),
        ],
    )
    def k(part_hbm, q_hbm, src_hbm, rev_hbm, d_out,
          stage, iv1, iv2, gbuf, rbuf, sem_i, sem_g, sem_r, sem_w):
        c = lax.axis_index("c")
        s = lax.axis_index("s")
        wid = c * 16 + s
        base = wid * ep

        def crows(r0, nr):
            def combine(rr, rows):
                pltpu.sync_copy(part_hbm.at[0, pl.ds(rr, rows)],
                                gbuf.at[0, pl.ds(0, rows)])
                pltpu.sync_copy(part_hbm.at[1, pl.ds(rr, rows)],
                                rbuf.at[0, pl.ds(0, rows)])

                def row(r, carry2):
                    for q in range(8):
                        sl = pl.ds(q * 16, 16)
                        gbuf[0, r, sl] = gbuf[0, r, sl] + rbuf[0, r, sl]
                    return carry2

                lax.fori_loop(0, rows, row, 0)
                pltpu.sync_copy(gbuf.at[0, pl.ds(0, rows)],
                                stage.at[pl.ds(rr, rows)])

            def cchunk(m, carry):
                combine(r0 + m * CROWS, CROWS)
                return carry

            lax.fori_loop(0, nr // CROWS, cchunk, 0)
            rem = nr % CROWS
            if rem:
                combine(r0 + nr - rem, rem)

        _for_tile_rows(s, N, crows)
        plsc.subcore_barrier()

        def idx_load(sc, slot):
            o = pl.ds(base + sc * SUPER, SUPER)
            pltpu.async_copy(src_hbm.at[o],
                             iv1.at[pl.ds(slot * SUPER, SUPER)],
                             sem_i.at[slot])
            pltpu.async_copy(rev_hbm.at[o],
                             iv2.at[pl.ds(slot * SUPER, SUPER)],
                             sem_i.at[slot])

        def idx_wait(slot):
            pltpu.make_async_copy(
                src_hbm.at[pl.ds(base, SUPER)],
                iv1.at[pl.ds(slot * SUPER, SUPER)], sem_i.at[slot]).wait()
            pltpu.make_async_copy(
                rev_hbm.at[pl.ds(base, SUPER)],
                iv2.at[pl.ds(slot * SUPER, SUPER)], sem_i.at[slot]).wait()

        def issue(sc, slot, w, b):
            o = slot * SUPER + w * CH
            pltpu.async_copy(stage.at[iv1.at[pl.ds(o, CH)]], gbuf.at[b],
                             sem_g.at[b])
            pltpu.async_copy(q_hbm.at[iv2.at[pl.ds(o, CH)]], rbuf.at[b],
                             sem_r.at[b])

        def wait_reads(b):
            pltpu.make_async_copy(stage.at[pl.ds(0, CH)], gbuf.at[b],
                                  sem_g.at[b]).wait()
            pltpu.make_async_copy(q_hbm.at[pl.ds(0, CH)], rbuf.at[b],
                                  sem_r.at[b]).wait()

        def drain_write(b):
            pltpu.make_async_copy(gbuf.at[b], d_out.at[pl.ds(0, CH)],
                                  sem_w.at[b]).wait()

        idx_load(0, 0)
        for sc in range(nsc):
            slot = sc % 2
            idx_wait(slot)
            if sc + 1 < nsc:
                idx_load(sc + 1, 1 - slot)
            issue(sc, slot, 0, 0)
            issue(sc, slot, 1, 1)

            def chunk(w, carry):
                b = w % NBUF
                wait_reads(b)

                @pl.when(jnp.logical_and(w >= 1, w < npc - 2))
                def _():
                    drain_write((w + 2) % NBUF)
                    issue(sc, slot, w + 2, (w + 2) % NBUF)

                @pl.when(w == 0)
                def _():
                    issue(sc, slot, 2, 2 % NBUF)

                def row(r, carry2):
                    for q in range(8):
                        sl = pl.ds(q * 16, 16)
                        gbuf[b, r, sl] = gbuf[b, r, sl] - rbuf[b, r, sl]
                    return carry2

                lax.fori_loop(0, CH, row, 0)
                off = base + sc * SUPER + w * CH
                pltpu.async_copy(gbuf.at[b], d_out.at[pl.ds(off, CH)],
                                 sem_w.at[b])
                return carry

            lax.fori_loop(0, npc, chunk, 0)
            for b in range(NBUF):
                drain_write(b)

    return k(part, q, src, rev)


def kernel(x, edge_index, edge_attr, rev_edge_index, W_i, b_i, W_h, b_h,
           W_o, b_o):
    N, DF = x.shape
    E = edge_attr.shape[0]
    depth = 5

    src = edge_index[0]
    dst = edge_index[1]
    zeros_n = jnp.zeros((N, 128), F32)

    W_ix = W_i[:, :DF]
    W_ie = W_i[:, DF:]

    xp = _tc_linear(x, W_ix, b_i, 1000)
    eh = _tc_linear(edge_attr, W_ie, jnp.zeros((128,), F32), 2000)
    h0 = _sc_h0(xp, eh, src, N, E)

    d = None
    for _ in range(1, depth):
        q = _tc_relu_matmul(h0, d, b_h, W_h, 2000)
        part = _sc_scatter(q, src, dst, h0, b_h, zeros_n, N, E,
                           relu_h0d=False)
        d = _sc_assemble(part, q, src, rev_edge_index, N, E)

    mn2 = _sc_scatter(d, src, dst, h0, b_h, zeros_n, N, E, relu_h0d=True)
    return _tc_final(x, mn2, W_o, b_o, 1000)
